# baseline (speedup 1.0000x reference)
import jax
import jax.numpy as jnp
from jax.experimental import pallas as pl
from jax.experimental.pallas import tpu as pltpu

MAX_LEN = 8192
DIM = 1024
BLOCK_ROWS = 256


def _body(t_ref, pe_ref, out_ref):
    i = pl.program_id(0)
    rows = jax.lax.broadcasted_iota(jnp.int32, (BLOCK_ROWS, 1), 0) + i * BLOCK_ROWS
    out_ref[...] = jnp.where(rows < t_ref[0], pe_ref[...], 0.0)


def kernel(pe, T):
    t_arr = jnp.asarray(T, dtype=jnp.int32).reshape((1,))
    n_blocks = MAX_LEN // BLOCK_ROWS
    out = pl.pallas_call(
        _body,
        grid=(n_blocks,),
        in_specs=[
            pl.BlockSpec(memory_space=pltpu.SMEM),
            pl.BlockSpec((BLOCK_ROWS, DIM), lambda i: (i, 0)),
        ],
        out_specs=pl.BlockSpec((BLOCK_ROWS, DIM), lambda i: (i, 0)),
        out_shape=jax.ShapeDtypeStruct((MAX_LEN, DIM), jnp.float32),
    )(t_arr, pe)
    return out[None, :, :]


# speedup vs baseline: 1.4645x; 1.4645x over previous
import jax
import jax.numpy as jnp
from jax.experimental import pallas as pl
from jax.experimental.pallas import tpu as pltpu

MAX_LEN = 8192
DIM = 1024
BLOCK_ROWS = 1024


def _body(t_ref, pe_ref, out_ref):
    i = pl.program_id(0)
    rows = jax.lax.broadcasted_iota(jnp.int32, (BLOCK_ROWS, 1), 0) + i * BLOCK_ROWS
    out_ref[...] = jnp.where(rows < t_ref[0], pe_ref[...], 0.0)


def kernel(pe, T):
    t_arr = jnp.asarray(T, dtype=jnp.int32).reshape((1,))
    n_blocks = MAX_LEN // BLOCK_ROWS
    out = pl.pallas_call(
        _body,
        grid=(n_blocks,),
        in_specs=[
            pl.BlockSpec(memory_space=pltpu.SMEM),
            pl.BlockSpec((BLOCK_ROWS, DIM), lambda i: (i, 0)),
        ],
        out_specs=pl.BlockSpec((BLOCK_ROWS, DIM), lambda i: (i, 0)),
        out_shape=jax.ShapeDtypeStruct((MAX_LEN, DIM), jnp.float32),
    )(t_arr, pe)
    return out[None, :, :]


# speedup vs baseline: 1.5645x; 1.0683x over previous
import jax
import jax.numpy as jnp
from jax.experimental import pallas as pl
from jax.experimental.pallas import tpu as pltpu

MAX_LEN = 8192
DIM = 1024
BLOCK_ROWS = 2048


def _body(t_ref, pe_ref, out_ref):
    i = pl.program_id(0)
    rows = jax.lax.broadcasted_iota(jnp.int32, (BLOCK_ROWS, 1), 0) + i * BLOCK_ROWS
    out_ref[...] = jnp.where(rows < t_ref[0], pe_ref[...], 0.0)


def kernel(pe, T):
    t_arr = jnp.asarray(T, dtype=jnp.int32).reshape((1,))
    n_blocks = MAX_LEN // BLOCK_ROWS
    out = pl.pallas_call(
        _body,
        grid=(n_blocks,),
        in_specs=[
            pl.BlockSpec(memory_space=pltpu.SMEM),
            pl.BlockSpec((BLOCK_ROWS, DIM), lambda i: (i, 0)),
        ],
        out_specs=pl.BlockSpec((BLOCK_ROWS, DIM), lambda i: (i, 0)),
        out_shape=jax.ShapeDtypeStruct((MAX_LEN, DIM), jnp.float32),
    )(t_arr, pe)
    return out[None, :, :]
